# Initial kernel scaffold; baseline (speedup 1.0000x reference)
import jax
import jax.numpy as jnp
from jax.experimental import pallas as pl


def kernel(training_set, test_set):
    raise NotImplementedError("write your pallas kernel here")


# speedup vs baseline: 425.6178x; 425.6178x over previous
import functools

import jax
import jax.numpy as jnp
from jax.experimental import pallas as pl
from jax.experimental.pallas import tpu as pltpu

N_TRAIN = 20000
W = 256
N_TEST = 1024
BC = 2048
N_PAD = 20480
N_TILES = N_PAD // BC
CLASS_SIZE = 2000
NUM_CLASSES = 10
MAX_DISTANCE = 256
SHIFT = 32768.0
BIG = 3.0e7


def _merge_top3(vals):
    def min_all(vs):
        m = vs[0]
        for v in vs[1:]:
            m = jnp.minimum(m, v)
        return m

    s1 = min_all(vals)
    vals2 = [jnp.where(v == s1, BIG, v) for v in vals]
    s2 = min_all(vals2)
    vals3 = [jnp.where(v == s2, BIG, v) for v in vals2]
    s3 = min_all(vals3)
    return s1, s2, s3


def _body(test_ref, train_ref, out_ref, r1, r2, r3):
    t = pl.program_id(0)
    dot = jax.lax.dot_general(
        test_ref[...], train_ref[...],
        (((1,), (1,)), ((), ())),
        preferred_element_type=jnp.float32,
    )
    ham = (float(W) - dot) * 0.5
    col_i = t * BC + jax.lax.broadcasted_iota(jnp.int32, (N_TEST, BC), 1)
    col = col_i.astype(jnp.float32)
    keys = ham * SHIFT + col
    keys = jnp.where(col >= float(N_TRAIN), BIG, keys)

    m1 = jnp.min(keys, axis=1, keepdims=True)
    k2 = jnp.where(keys == m1, BIG, keys)
    m2 = jnp.min(k2, axis=1, keepdims=True)
    k3 = jnp.where(k2 == m2, BIG, k2)
    m3 = jnp.min(k3, axis=1, keepdims=True)

    @pl.when(t == 0)
    def _():
        r1[...] = m1
        r2[...] = m2
        r3[...] = m3

    @pl.when(t > 0)
    def _():
        s1, s2, s3 = _merge_top3([r1[...], r2[...], r3[...], m1, m2, m3])
        r1[...] = s1
        r2[...] = s2
        r3[...] = s3

    @pl.when(t == N_TILES - 1)
    def _():
        def decode(key_f):
            ki = key_f.astype(jnp.int32)
            dist = ki >> 15
            idx = ki & 32767
            lab = jnp.zeros_like(idx)
            for c in range(1, NUM_CLASSES):
                lab = lab + (idx >= c * CLASS_SIZE).astype(jnp.int32)
            return jnp.where(dist < MAX_DISTANCE, lab, 0)

        l1 = decode(r1[...])
        l2 = decode(r2[...])
        l3 = decode(r3[...])
        maj = jnp.where(
            (l1 == l2) | (l1 == l3), l1,
            jnp.where(l2 == l3, l2, jnp.minimum(l1, jnp.minimum(l2, l3))),
        )
        out_ref[...] = maj


@jax.jit
def _knn(test_pm, train_pm):
    out = pl.pallas_call(
        _body,
        grid=(N_TILES,),
        in_specs=[
            pl.BlockSpec((N_TEST, W), lambda t: (0, 0)),
            pl.BlockSpec((BC, W), lambda t: (t, 0)),
        ],
        out_specs=pl.BlockSpec((N_TEST, 1), lambda t: (0, 0)),
        out_shape=jax.ShapeDtypeStruct((N_TEST, 1), jnp.int32),
        scratch_shapes=[pltpu.VMEM((N_TEST, 1), jnp.float32)] * 3,
        compiler_params=pltpu.CompilerParams(
            dimension_semantics=("arbitrary",),
        ),
    )(test_pm, train_pm)
    return out.reshape(N_TEST)


def kernel(training_set, test_set):
    train_pm = (1 - 2 * training_set).astype(jnp.bfloat16)
    train_pm = jnp.pad(train_pm, ((0, N_PAD - N_TRAIN), (0, 0)))
    test_pm = (1 - 2 * test_set).astype(jnp.bfloat16)
    return _knn(test_pm, train_pm)


# speedup vs baseline: 487.1402x; 1.1445x over previous
import functools

import jax
import jax.numpy as jnp
from jax.experimental import pallas as pl
from jax.experimental.pallas import tpu as pltpu

N_TRAIN = 20000
W = 256
N_TEST = 1024
BC = 2048
N_PAD = 20480
N_TILES = N_PAD // BC
LANES = 128
CHUNKS = BC // LANES
CLASS_SIZE = 2000
NUM_CLASSES = 10
MAX_DISTANCE = 256
SCALE = 16384.0
BIG = 3.0e7


def _body(test_ref, train_ref, colv_ref, out_ref, t1, t2, t3):
    t = pl.program_id(0)
    dot = jax.lax.dot_general(
        test_ref[...], train_ref[...],
        (((1,), (1,)), ((), ())),
        preferred_element_type=jnp.float32,
    )
    keys = dot + colv_ref[...]

    @pl.when(t == 0)
    def _():
        t1[...] = jnp.full((N_TEST, LANES), BIG, jnp.float32)
        t2[...] = jnp.full((N_TEST, LANES), BIG, jnp.float32)
        t3[...] = jnp.full((N_TEST, LANES), BIG, jnp.float32)

    a1, a2, a3 = t1[...], t2[...], t3[...]
    for c in range(CHUNKS):
        x = keys[:, c * LANES:(c + 1) * LANES]
        lo = jnp.minimum(a1, x)
        hi = jnp.maximum(a1, x)
        a1 = lo
        lo = jnp.minimum(a2, hi)
        hi = jnp.maximum(a2, hi)
        a2 = lo
        a3 = jnp.minimum(a3, hi)
    t1[...] = a1
    t2[...] = a2
    t3[...] = a3

    @pl.when(t == N_TILES - 1)
    def _():
        cand = jnp.concatenate([a1, a2, a3], axis=1)
        m1 = jnp.min(cand, axis=1, keepdims=True)
        c2 = jnp.where(cand == m1, BIG, cand)
        m2 = jnp.min(c2, axis=1, keepdims=True)
        c3 = jnp.where(c2 == m2, BIG, c2)
        m3 = jnp.min(c3, axis=1, keepdims=True)

        def decode(key_f):
            ki = key_f.astype(jnp.int32)
            dist = ki >> 15
            idx = ki & 32767
            lab = (idx * 8389) >> 24
            return jnp.where(dist < MAX_DISTANCE, lab, 0)

        l1, l2, l3 = decode(m1), decode(m2), decode(m3)
        out_ref[...] = jnp.where(
            (l1 == l2) | (l1 == l3), l1,
            jnp.where(l2 == l3, l2, jnp.minimum(l1, jnp.minimum(l2, l3))),
        )


@jax.jit
def _knn(test_in, train_in, colvec):
    out = pl.pallas_call(
        _body,
        grid=(N_TILES,),
        in_specs=[
            pl.BlockSpec((N_TEST, W), lambda t: (0, 0)),
            pl.BlockSpec((BC, W), lambda t: (t, 0)),
            pl.BlockSpec((1, BC), lambda t: (0, t)),
        ],
        out_specs=pl.BlockSpec((N_TEST, 1), lambda t: (0, 0)),
        out_shape=jax.ShapeDtypeStruct((N_TEST, 1), jnp.int32),
        scratch_shapes=[pltpu.VMEM((N_TEST, LANES), jnp.float32)] * 3,
        compiler_params=pltpu.CompilerParams(
            dimension_semantics=("arbitrary",),
        ),
    )(test_in, train_in, colvec)
    return out.reshape(N_TEST)


def kernel(training_set, test_set):
    test_in = ((2 * test_set - 1) * 16384).astype(jnp.bfloat16)
    train_in = (1 - 2 * training_set).astype(jnp.bfloat16)
    train_in = jnp.pad(train_in, ((0, N_PAD - N_TRAIN), (0, 0)))
    j = jnp.arange(N_PAD, dtype=jnp.float32)
    colvec = jnp.where(j < N_TRAIN, SCALE * W + j, 2.5e7 + j).reshape(1, N_PAD)
    return _knn(test_in, train_in, colvec)
